# baseline (speedup 1.0000x reference)
import jax
import jax.numpy as jnp
from jax.experimental import pallas as pl
from jax.experimental.pallas import tpu as pltpu
from jax.experimental.pallas import tpu_sc as plsc


def _sc_gather(table, idx_flat):
    B = idx_flat.shape[0]
    D = table.shape[1]
    NW = 32
    b_per_w = B // NW
    chunk = b_per_w
    while chunk * D * 4 > 256 * 1024:
        chunk //= 2
    n_chunks = b_per_w // chunk
    mesh = plsc.VectorSubcoreMesh(core_axis_name="c", subcore_axis_name="s")

    @pl.kernel(
        out_type=jax.ShapeDtypeStruct((B, D), table.dtype),
        mesh=mesh,
        scratch_types=[
            pltpu.VMEM((chunk,), jnp.int32),
            pltpu.VMEM((chunk, D), table.dtype),
            pltpu.SemaphoreType.DMA,
        ],
    )
    def gather_kernel(tab_hbm, idx_hbm, out_hbm, idx_v, rows_v, sem):
        wid = jax.lax.axis_index("s") * 2 + jax.lax.axis_index("c")
        base = wid * b_per_w

        @pl.loop(0, n_chunks)
        def _(ci):
            off = base + ci * chunk
            pltpu.sync_copy(idx_hbm.at[pl.ds(off, chunk)], idx_v)
            pltpu.async_copy(tab_hbm.at[idx_v], rows_v, sem).wait()
            pltpu.sync_copy(rows_v, out_hbm.at[pl.ds(off, chunk)])

    return gather_kernel(table, idx_flat)


def _ln_body(tok_ref, pos_ref, g_ref, b_ref, o_ref):
    x = tok_ref[...] + pos_ref[...]
    mean = jnp.mean(x, axis=-1, keepdims=True)
    cent = x - mean
    var = jnp.mean(cent * cent, axis=-1, keepdims=True)
    y = cent * jax.lax.rsqrt(var + 1e-5) * g_ref[...] + b_ref[...]
    o_ref[...] = y.astype(jnp.bfloat16)


def _ln(tok_emb, pos_table, gamma, beta, row_offset):
    T, D = tok_emb.shape
    ROWS = min(256, T)
    off = row_offset // ROWS
    return pl.pallas_call(
        _ln_body,
        grid=(T // ROWS,),
        in_specs=[
            pl.BlockSpec((ROWS, D), lambda i: (i, 0)),
            pl.BlockSpec((ROWS, D), lambda i, o=off: (i + o, 0)),
            pl.BlockSpec((1, D), lambda i: (0, 0)),
            pl.BlockSpec((1, D), lambda i: (0, 0)),
        ],
        out_specs=pl.BlockSpec((ROWS, D), lambda i: (i, 0)),
        out_shape=jax.ShapeDtypeStruct((T, D), jnp.bfloat16),
        compiler_params=pltpu.CompilerParams(
            dimension_semantics=("parallel",)),
    )(tok_emb, pos_table, gamma.reshape(1, D), beta.reshape(1, D))


def _mm_half_body(w_ref, x_ref, o_ref):
    w = w_ref[...].astype(jnp.bfloat16)
    acc = jax.lax.dot_general(
        w, x_ref[...], (((1,), (1,)), ((), ())),
        preferred_element_type=jnp.float32,
    )
    vt, t2 = acc.shape
    o_ref[...] = acc.reshape(vt, 1, t2 // 128, 128)


def _mm_half_body_alias(w_ref, x_ref, p_ref, o_ref):
    del p_ref
    _mm_half_body(w_ref, x_ref, o_ref)


def _lm_head_half(x_bf16, W_lm, half, prev=None):
    T2, D = x_bf16.shape
    V = W_lm.shape[0]
    VT = 1024
    NTH = T2 // 128
    in_specs = [
        pl.BlockSpec((VT, D), lambda j: (j, 0)),
        pl.BlockSpec((T2, D), lambda j: (0, 0)),
    ]
    args = [W_lm, x_bf16]
    aliases = {}
    body = _mm_half_body
    if prev is not None:
        in_specs.append(pl.BlockSpec(memory_space=pl.ANY))
        args.append(prev)
        aliases = {2: 0}
        body = _mm_half_body_alias
    return pl.pallas_call(
        body,
        grid=(pl.cdiv(V, VT),),
        in_specs=in_specs,
        out_specs=pl.BlockSpec((VT, 1, NTH, 128), lambda j, h=half: (j, h, 0, 0)),
        out_shape=jax.ShapeDtypeStruct((V, 2, NTH, 128), jnp.float32),
        input_output_aliases=aliases,
        compiler_params=pltpu.CompilerParams(
            dimension_semantics=("parallel",)),
    )(*args)


def kernel(idx, tok_table, pos_table, ln_gamma, ln_beta, W_lm):
    B, T = idx.shape
    D = tok_table.shape[1]
    V = W_lm.shape[0]
    T2 = T // 2
    idx_flat = idx.reshape(B * T).astype(jnp.int32)
    tok0 = _sc_gather(tok_table, idx_flat[:T2])
    tok1 = _sc_gather(tok_table, idx_flat[T2:])
    x0 = _ln(tok0, pos_table, ln_gamma, ln_beta, 0)
    x1 = _ln(tok1, pos_table, ln_gamma, ln_beta, T2)
    out0 = _lm_head_half(x0, W_lm, 0)
    out1 = _lm_head_half(x1, W_lm, 1, out0)
    nt = T // 128
    return out1.reshape(V, nt, 128).transpose(1, 2, 0).reshape(B, T, V)


# speedup vs baseline: 1.2874x; 1.2874x over previous
import jax
import jax.numpy as jnp
from jax.experimental import pallas as pl
from jax.experimental.pallas import tpu as pltpu
from jax.experimental.pallas import tpu_sc as plsc


def _sc_gather(table, idx_flat):
    B = idx_flat.shape[0]
    D = table.shape[1]
    NW = 32
    b_per_w = B // NW
    chunk = b_per_w
    while chunk * D * 4 > 256 * 1024:
        chunk //= 2
    n_chunks = b_per_w // chunk
    mesh = plsc.VectorSubcoreMesh(core_axis_name="c", subcore_axis_name="s")

    @pl.kernel(
        out_type=jax.ShapeDtypeStruct((B, D), table.dtype),
        mesh=mesh,
        scratch_types=[
            pltpu.VMEM((chunk,), jnp.int32),
            pltpu.VMEM((chunk, D), table.dtype),
            pltpu.SemaphoreType.DMA,
        ],
    )
    def gather_kernel(tab_hbm, idx_hbm, out_hbm, idx_v, rows_v, sem):
        wid = jax.lax.axis_index("s") * 2 + jax.lax.axis_index("c")
        base = wid * b_per_w

        @pl.loop(0, n_chunks)
        def _(ci):
            off = base + ci * chunk
            pltpu.sync_copy(idx_hbm.at[pl.ds(off, chunk)], idx_v)
            pltpu.async_copy(tab_hbm.at[idx_v], rows_v, sem).wait()
            pltpu.sync_copy(rows_v, out_hbm.at[pl.ds(off, chunk)])

    return gather_kernel(table, idx_flat)


def _ln_body(tok_ref, pos_ref, g_ref, b_ref, o_ref):
    x = tok_ref[...] + pos_ref[...]
    mean = jnp.mean(x, axis=-1, keepdims=True)
    cent = x - mean
    var = jnp.mean(cent * cent, axis=-1, keepdims=True)
    y = cent * jax.lax.rsqrt(var + 1e-5) * g_ref[...] + b_ref[...]
    o_ref[...] = y.astype(jnp.bfloat16)


def _ln(tok_emb, pos_emb, gamma, beta):
    T, D = tok_emb.shape
    ROWS = min(256, T)
    return pl.pallas_call(
        _ln_body,
        grid=(T // ROWS,),
        in_specs=[
            pl.BlockSpec((ROWS, D), lambda i: (i, 0)),
            pl.BlockSpec((ROWS, D), lambda i: (i, 0)),
            pl.BlockSpec((1, D), lambda i: (0, 0)),
            pl.BlockSpec((1, D), lambda i: (0, 0)),
        ],
        out_specs=pl.BlockSpec((ROWS, D), lambda i: (i, 0)),
        out_shape=jax.ShapeDtypeStruct((T, D), jnp.bfloat16),
        compiler_params=pltpu.CompilerParams(
            dimension_semantics=("parallel",)),
    )(tok_emb, pos_emb, gamma.reshape(1, D), beta.reshape(1, D))


def _mm_body(w_ref, x_ref, o_ref):
    w = w_ref[...].astype(jnp.bfloat16)
    acc = jax.lax.dot_general(
        w, x_ref[...], (((1,), (1,)), ((), ())),
        preferred_element_type=jnp.float32,
    )
    vt, t = acc.shape
    o_ref[...] = acc.reshape(vt * (t // 128), 128)


def _lm_head(x_bf16, W_lm):
    T, D = x_bf16.shape
    V = W_lm.shape[0]
    VT = 1024
    NT = T // 128
    return pl.pallas_call(
        _mm_body,
        grid=(pl.cdiv(V, VT),),
        in_specs=[
            pl.BlockSpec((VT, D), lambda j: (j, 0)),
            pl.BlockSpec((T, D), lambda j: (0, 0)),
        ],
        out_specs=pl.BlockSpec((VT * NT, 128), lambda j: (j, 0)),
        out_shape=jax.ShapeDtypeStruct((V * NT, 128), jnp.float32),
        compiler_params=pltpu.CompilerParams(
            dimension_semantics=("parallel",)),
    )(W_lm, x_bf16)


def kernel(idx, tok_table, pos_table, ln_gamma, ln_beta, W_lm):
    B, T = idx.shape
    D = tok_table.shape[1]
    V = W_lm.shape[0]
    idx_flat = idx.reshape(B * T).astype(jnp.int32)
    tok_emb = _sc_gather(tok_table, idx_flat)
    x = _ln(tok_emb, pos_table[:T], ln_gamma, ln_beta)
    logits_t = _lm_head(x, W_lm)
    nt = T // 128
    return logits_t.reshape(V, nt, 128).transpose(1, 2, 0).reshape(B, T, V)
